# Initial kernel scaffold; baseline (speedup 1.0000x reference)
import jax
import jax.numpy as jnp
from jax.experimental import pallas as pl


def kernel(cls_logits, bbox_regression, anchors, gt_boxes, gt_labels):
    raise NotImplementedError("write your pallas kernel here")


# speedup vs baseline: 3.4651x; 3.4651x over previous
import functools

import jax
import jax.numpy as jnp
from jax.experimental import pallas as pl
from jax.experimental.pallas import tpu as pltpu

B, A, G, C = 8, 24576, 64, 81
IOU_THRES = 0.45
NEG_TO_POS_RATIO = 3
BLK = 2048
NB = A // BLK


def _aabb_cols(q):
    x1 = jnp.minimum(jnp.minimum(q[:, 0:1], q[:, 2:3]),
                     jnp.minimum(q[:, 4:5], q[:, 6:7]))
    x2 = jnp.maximum(jnp.maximum(q[:, 0:1], q[:, 2:3]),
                     jnp.maximum(q[:, 4:5], q[:, 6:7]))
    y1 = jnp.minimum(jnp.minimum(q[:, 1:2], q[:, 3:4]),
                     jnp.minimum(q[:, 5:6], q[:, 7:8]))
    y2 = jnp.maximum(jnp.maximum(q[:, 1:2], q[:, 3:4]),
                     jnp.maximum(q[:, 5:6], q[:, 7:8]))
    return x1, y1, x2, y2


def _aabb_rows(qt):
    x1 = jnp.minimum(jnp.minimum(qt[0:1], qt[2:3]), jnp.minimum(qt[4:5], qt[6:7]))
    x2 = jnp.maximum(jnp.maximum(qt[0:1], qt[2:3]), jnp.maximum(qt[4:5], qt[6:7]))
    y1 = jnp.minimum(jnp.minimum(qt[1:2], qt[3:4]), jnp.minimum(qt[5:6], qt[7:8]))
    y2 = jnp.maximum(jnp.maximum(qt[1:2], qt[3:4]), jnp.maximum(qt[5:6], qt[7:8]))
    return x1, y1, x2, y2


def _match_body(gt_t_ref, an_ref, matched_ref, bestv_ref, besti_ref):
    j = pl.program_id(1)
    gt_t = gt_t_ref[0]
    an = an_ref[0]
    gx1, gy1, gx2, gy2 = _aabb_rows(gt_t)
    ax1, ay1, ax2, ay2 = _aabb_cols(an)
    ix1 = jnp.maximum(gx1, ax1)
    iy1 = jnp.maximum(gy1, ay1)
    ix2 = jnp.minimum(gx2, ax2)
    iy2 = jnp.minimum(gy2, ay2)
    inter = jnp.clip(ix2 - ix1, 0.0) * jnp.clip(iy2 - iy1, 0.0)
    area_g = (gx2 - gx1) * (gy2 - gy1)
    area_a = (ax2 - ax1) * (ay2 - ay1)
    union = area_g + area_a - inter
    iou = inter / jnp.maximum(union, 1e-8)

    m = jnp.argmax(iou, axis=1).astype(jnp.int32)
    maxv = jnp.max(iou, axis=1)
    matched_ref[0, 0, :] = jnp.where(maxv >= IOU_THRES, m, -1)

    rowmax = jnp.max(iou, axis=0)
    rowarg = jnp.argmax(iou, axis=0).astype(jnp.int32) + j * BLK

    @pl.when(j == 0)
    def _():
        bestv_ref[0, 0, :] = rowmax
        besti_ref[0, 0, :] = rowarg

    @pl.when(j > 0)
    def _():
        cur = bestv_ref[0, 0, :]
        upd = rowmax > cur
        bestv_ref[0, 0, :] = jnp.where(upd, rowmax, cur)
        besti_ref[0, 0, :] = jnp.where(upd, rowarg, besti_ref[0, 0, :])


def _loss_body(matched_ref, besti_ref, gtcat_ref, an_ref, bb_ref, cls_ref,
               closs_ref, neg_ref, acc_ref):
    b = pl.program_id(0)
    j = pl.program_id(1)
    m = matched_ref[0, 0, :]
    best = besti_ref[0]
    ids = j * BLK + jax.lax.broadcasted_iota(jnp.int32, (BLK, 1), 0)
    eq = best == ids
    g_iota = jax.lax.broadcasted_iota(jnp.int32, (BLK, G), 1)
    forced = jnp.max(jnp.where(eq, g_iota, -1), axis=1)
    m2 = jnp.where(forced >= 0, forced, m)
    fg = m2 >= 0
    safe = jnp.clip(m2, 0, G - 1)

    onehot = (safe[:, None] == jax.lax.broadcasted_iota(jnp.int32, (1, G), 1)
              ).astype(jnp.float32)
    gtcat = gtcat_ref[0]
    gath = jnp.dot(onehot, gtcat, preferred_element_type=jnp.float32)
    mb = gath[:, 0:8]
    mlab = gath[:, 8].astype(jnp.int32)

    an = an_ref[0]
    ax1, ay1, ax2, ay2 = _aabb_cols(an)
    aw = jnp.maximum(ax2 - ax1, 1e-6)
    ah = jnp.maximum(ay2 - ay1, 1e-6)
    denom = jnp.concatenate([aw, ah, aw, ah, aw, ah, aw, ah], axis=1)
    target = (mb - an) / denom
    d = bb_ref[0] - target
    ad = jnp.abs(d)
    sl1 = jnp.sum(jnp.where(ad < 1.0, 0.5 * d * d, ad - 0.5), axis=1)
    bpart = jnp.sum(jnp.where(fg, sl1, 0.0))

    x = cls_ref[0]
    mx = jnp.max(x, axis=1, keepdims=True)
    lse = jnp.log(jnp.sum(jnp.exp(x - mx), axis=1, keepdims=True)) + mx
    tgt = jnp.where(fg, mlab, C - 1)
    oh81 = tgt[:, None] == jax.lax.broadcasted_iota(jnp.int32, (1, C), 1)
    xt = jnp.sum(jnp.where(oh81, x, 0.0), axis=1)
    closs = lse[:, 0] - xt
    closs_ref[0, 0, :] = closs
    neg_ref[0, 0, :] = jnp.where(fg, -1.0, closs)

    lane = jax.lax.broadcasted_iota(jnp.int32, (1, 128), 1)
    vec = jnp.where(lane == 0, bpart, 0.0)

    @pl.when((b == 0) & (j == 0))
    def _():
        acc_ref[...] = vec

    @pl.when((b > 0) | (j > 0))
    def _():
        acc_ref[...] = acc_ref[...] + vec


def _select_body(closs_ref, neg_ref, acc_ref, res_ref):
    b = pl.program_id(0)
    neg = neg_ref[0]
    cls = closs_ref[0]
    fgm = neg < 0.0
    nfg = jnp.sum(fgm.astype(jnp.int32))
    num_bg = A - nfg
    kk = NEG_TO_POS_RATIO * nfg
    k1 = jnp.minimum(kk, num_bg)

    v_int = jnp.where(fgm, -1, jax.lax.bitcast_convert_type(neg, jnp.int32))

    def bs_body(_, lohi):
        lo, hi = lohi
        mid = jax.lax.div(lo + hi, 2)
        cnt = jnp.sum((v_int > mid).astype(jnp.int32))
        small = cnt < k1
        return jnp.where(small, lo, mid + 1), jnp.where(small, mid, hi)

    lo, _hi = jax.lax.fori_loop(0, 31, bs_body,
                                (jnp.int32(0), jnp.int32(2**31 - 1)))
    t = lo
    gt_t = v_int > t
    cnt_gt = jnp.sum(gt_t.astype(jnp.int32))
    sum_gt = jnp.sum(jnp.where(gt_t, neg, 0.0))
    t_f = jax.lax.bitcast_convert_type(t, jnp.float32)
    s1 = sum_gt + (k1 - cnt_gt).astype(jnp.float32) * jnp.where(cnt_gt < k1, t_f, 0.0)

    extra = jnp.clip(kk - num_bg, 0, nfg)
    idx = (jax.lax.broadcasted_iota(jnp.int32, neg.shape, 0) * 128 +
           jax.lax.broadcasted_iota(jnp.int32, neg.shape, 1))

    def bs2_body(_, lohi):
        lo, hi = lohi
        mid = jax.lax.div(lo + hi, 2)
        cnt = jnp.sum((fgm & (idx < mid)).astype(jnp.int32))
        enough = cnt >= extra
        return jnp.where(enough, lo, mid + 1), jnp.where(enough, mid, hi)

    p, _p2 = jax.lax.fori_loop(0, 16, bs2_body, (jnp.int32(0), jnp.int32(A)))
    s2 = jnp.sum(jnp.where(fgm & (idx < p), cls, 0.0))

    cneg_row = s1 + s2
    cfg_row = jnp.sum(jnp.where(fgm, cls, 0.0))

    lane = jax.lax.broadcasted_iota(jnp.int32, (1, 128), 1)
    vec = (jnp.where(lane == 0, nfg.astype(jnp.float32), 0.0) +
           jnp.where(lane == 1, cfg_row, 0.0) +
           jnp.where(lane == 2, cneg_row, 0.0))

    @pl.when(b == 0)
    def _():
        res_ref[...] = vec + jnp.where(lane == 3, jnp.sum(acc_ref[...]), 0.0)

    @pl.when(b > 0)
    def _():
        res_ref[...] = res_ref[...] + vec


@jax.jit
def kernel(cls_logits, bbox_regression, anchors, gt_boxes, gt_labels):
    gt_t = gt_boxes.transpose(0, 2, 1)
    gtcat = jnp.concatenate(
        [gt_boxes, gt_labels[..., None].astype(jnp.float32)], axis=-1)

    matched, bestv, besti = pl.pallas_call(
        _match_body,
        grid=(B, NB),
        in_specs=[
            pl.BlockSpec((1, 8, G), lambda b, j: (b, 0, 0)),
            pl.BlockSpec((1, BLK, 8), lambda b, j: (b, j, 0)),
        ],
        out_specs=[
            pl.BlockSpec((1, 1, BLK), lambda b, j: (b, 0, j)),
            pl.BlockSpec((1, 1, G), lambda b, j: (b, 0, 0)),
            pl.BlockSpec((1, 1, G), lambda b, j: (b, 0, 0)),
        ],
        out_shape=[
            jax.ShapeDtypeStruct((B, 1, A), jnp.int32),
            jax.ShapeDtypeStruct((B, 1, G), jnp.float32),
            jax.ShapeDtypeStruct((B, 1, G), jnp.int32),
        ],
    )(gt_t, anchors)

    closs, negv, acc = pl.pallas_call(
        _loss_body,
        grid=(B, NB),
        in_specs=[
            pl.BlockSpec((1, 1, BLK), lambda b, j: (b, 0, j)),
            pl.BlockSpec((1, 1, G), lambda b, j: (b, 0, 0)),
            pl.BlockSpec((1, G, 9), lambda b, j: (b, 0, 0)),
            pl.BlockSpec((1, BLK, 8), lambda b, j: (b, j, 0)),
            pl.BlockSpec((1, BLK, 8), lambda b, j: (b, j, 0)),
            pl.BlockSpec((1, BLK, C), lambda b, j: (b, j, 0)),
        ],
        out_specs=[
            pl.BlockSpec((1, 1, BLK), lambda b, j: (b, 0, j)),
            pl.BlockSpec((1, 1, BLK), lambda b, j: (b, 0, j)),
            pl.BlockSpec((1, 128), lambda b, j: (0, 0)),
        ],
        out_shape=[
            jax.ShapeDtypeStruct((B, 1, A), jnp.float32),
            jax.ShapeDtypeStruct((B, 1, A), jnp.float32),
            jax.ShapeDtypeStruct((1, 128), jnp.float32),
        ],
    )(matched, besti, gtcat, anchors, bbox_regression, cls_logits)

    closs2 = closs.reshape(B, A // 128, 128)
    negv2 = negv.reshape(B, A // 128, 128)

    res = pl.pallas_call(
        _select_body,
        grid=(B,),
        in_specs=[
            pl.BlockSpec((1, A // 128, 128), lambda b: (b, 0, 0)),
            pl.BlockSpec((1, A // 128, 128), lambda b: (b, 0, 0)),
            pl.BlockSpec((1, 128), lambda b: (0, 0)),
        ],
        out_specs=pl.BlockSpec((1, 128), lambda b: (0, 0)),
        out_shape=jax.ShapeDtypeStruct((1, 128), jnp.float32),
    )(closs2, negv2, acc)

    nfg_total = res[0, 0]
    cls_fg = res[0, 1]
    cls_neg = res[0, 2]
    bbox_total = res[0, 3]
    n = jnp.maximum(1.0, nfg_total)
    bbox_loss = bbox_total / n
    cls_total = (cls_fg + cls_neg) / n
    return bbox_loss + cls_total, bbox_loss, cls_total
